# Initial kernel scaffold; baseline (speedup 1.0000x reference)
import jax
import jax.numpy as jnp
from jax.experimental import pallas as pl


def kernel(embedding_table, flat_node_ids, segment_ids):
    raise NotImplementedError("write your pallas kernel here")


# speedup vs baseline: 2.9261x; 2.9261x over previous
import functools

import jax
import jax.numpy as jnp
from jax import lax
from jax.experimental import pallas as pl
from jax.experimental.pallas import tpu as pltpu
from jax.experimental.pallas import tpu_sc as plsc

N_NODES = 10000
D = 128
E = 320000
H = 20000

NUM_WORKERS = 32
SEG_W = 632
SLAB_ROWS = SEG_W + 16
C = 128
NLANE = 16
DGRP = D // NLANE

_MESH = plsc.VectorSubcoreMesh(core_axis_name="c", subcore_axis_name="s")


def _lower_bound(seg_hbm, target, idx16, segv16, sem):
    lanes = jnp.arange(16, dtype=jnp.int32)

    def rnd(_, carry):
        lo, n = carry
        step = (n + 15) // 16
        probes = lo + (lanes + 1) * step - 1
        hi_abs = jnp.minimum(lo + n, E)
        valid = probes < hi_abs
        idx16[...] = jnp.minimum(probes, E - 1)
        pltpu.async_copy(seg_hbm.at[idx16], segv16, sem).wait()
        less = valid & (segv16[...] < target)
        cnt = plsc.all_reduce_population_count(less)[0]
        return lo + cnt * step, step

    lo, _ = lax.fori_loop(0, 6, rnd, (jnp.int32(0), jnp.int32(E)))
    return lo


def _sc_body(emb_hbm, ids_hbm, seg_hbm, out_hbm,
             idx16, segv16, idsv, segv, rowsv, slab, sem):
    wid = lax.axis_index("s") * 2 + lax.axis_index("c")
    t_lo = jnp.minimum(wid * SEG_W, H - SEG_W)

    lo = _lower_bound(seg_hbm, t_lo, idx16, segv16, sem)
    hi = _lower_bound(seg_hbm, t_lo + SEG_W, idx16, segv16, sem)
    a_lo = (lo // 8) * 8
    a_hi = jnp.minimum(((hi + 7) // 8) * 8, E)
    nchunks = (a_hi - a_lo + C - 1) // C

    neg_inf = jnp.full((NLANE,), -jnp.inf, dtype=jnp.float32)

    def init_body(r, _):
        for d in range(DGRP):
            slab[r, pl.ds(d * NLANE, NLANE)] = neg_inf
        return 0

    lax.fori_loop(0, SLAB_ROWS, init_body, 0)

    def chunk_body(c, _):
        base = jnp.minimum(a_lo + c * C, E - C)
        pltpu.sync_copy(seg_hbm.at[pl.ds(base, C)], segv)
        pltpu.sync_copy(ids_hbm.at[pl.ds(base, C)], idsv)
        pltpu.async_copy(emb_hbm.at[idsv], rowsv, sem).wait()

        def mem_body(g, _):
            gbase = g * NLANE
            sv = segv[pl.ds(gbase, NLANE)]
            rv = jnp.clip(sv - t_lo, -1, SEG_W) + 8
            for j in range(NLANE):
                r = rv[j]
                i = gbase + j
                for d in range(DGRP):
                    sl = pl.ds(d * NLANE, NLANE)
                    slab[r, sl] = jnp.maximum(slab[r, sl], rowsv[i, sl])
            return 0

        lax.fori_loop(0, C // NLANE, mem_body, 0)
        return 0

    lax.fori_loop(0, nchunks, chunk_body, 0)

    pltpu.sync_copy(slab.at[pl.ds(8, SEG_W)], out_hbm.at[pl.ds(t_lo, SEG_W)])


@functools.partial(
    pl.kernel,
    out_type=jax.ShapeDtypeStruct((H, D), jnp.float32),
    mesh=_MESH,
    scratch_types=[
        pltpu.VMEM((16,), jnp.int32),
        pltpu.VMEM((16,), jnp.int32),
        pltpu.VMEM((C,), jnp.int32),
        pltpu.VMEM((C,), jnp.int32),
        pltpu.VMEM((C, D), jnp.float32),
        pltpu.VMEM((SLAB_ROWS, D), jnp.float32),
        pltpu.SemaphoreType.DMA,
    ],
    compiler_params=pltpu.CompilerParams(needs_layout_passes=False),
)
def _hyperedge_max(emb_hbm, ids_hbm, seg_hbm, out_hbm,
                   idx16, segv16, idsv, segv, rowsv, slab, sem):
    _sc_body(emb_hbm, ids_hbm, seg_hbm, out_hbm,
             idx16, segv16, idsv, segv, rowsv, slab, sem)


@jax.jit
def kernel(embedding_table, flat_node_ids, segment_ids):
    return _hyperedge_max(embedding_table, flat_node_ids, segment_ids)


# speedup vs baseline: 4.5945x; 1.5702x over previous
import functools

import jax
import jax.numpy as jnp
from jax import lax
from jax.experimental import pallas as pl
from jax.experimental.pallas import tpu as pltpu
from jax.experimental.pallas import tpu_sc as plsc

N_NODES = 10000
D = 128
E = 320000
H = 20000

NUM_WORKERS = 32
SEG_W = 632
SLAB_ROWS = SEG_W + 16
C = 128
NLANE = 16
DGRP = D // NLANE

_MESH = plsc.VectorSubcoreMesh(core_axis_name="c", subcore_axis_name="s")


def _lower_bound(seg_hbm, target, idx16, segv16, sem):
    lanes = jnp.arange(16, dtype=jnp.int32)

    def rnd(_, carry):
        lo, n = carry
        step = (n + 15) // 16
        probes = lo + (lanes + 1) * step - 1
        hi_abs = jnp.minimum(lo + n, E)
        valid = probes < hi_abs
        idx16[...] = jnp.minimum(probes, E - 1)
        pltpu.async_copy(seg_hbm.at[idx16], segv16, sem).wait()
        less = valid & (segv16[...] < target)
        cnt = plsc.all_reduce_population_count(less)[0]
        return lo + cnt * step, step

    lo, _ = lax.fori_loop(0, 6, rnd, (jnp.int32(0), jnp.int32(E)))
    return lo


def _sc_body(emb_hbm, ids_hbm, seg_hbm, out_hbm,
             idx16, segv16, idsv, segv, rowsv, slab, sem):
    wid = lax.axis_index("s") * 2 + lax.axis_index("c")
    t_lo = jnp.minimum(wid * SEG_W, H - SEG_W)

    lo = _lower_bound(seg_hbm, t_lo, idx16, segv16, sem)
    hi = _lower_bound(seg_hbm, t_lo + SEG_W, idx16, segv16, sem)
    a_lo = (lo // 8) * 8
    a_hi = jnp.minimum(((hi + 7) // 8) * 8, E)
    nchunks = (a_hi - a_lo + C - 1) // C

    neg_inf = jnp.full((NLANE,), -jnp.inf, dtype=jnp.float32)

    def init_body(r, _):
        for d in range(DGRP):
            slab[r, pl.ds(d * NLANE, NLANE)] = neg_inf
        return 0

    lax.fori_loop(0, SLAB_ROWS, init_body, 0)

    def chunk_body(c, carry):
        base = jnp.minimum(a_lo + c * C, E - C)
        pltpu.sync_copy(seg_hbm.at[pl.ds(base, C)], segv)
        pltpu.sync_copy(ids_hbm.at[pl.ds(base, C)], idsv)
        pltpu.async_copy(emb_hbm.at[idsv], rowsv, sem).wait()

        def mem_body(g, carry):
            gbase = g * NLANE
            sv = segv[pl.ds(gbase, NLANE)]
            rv = jnp.clip(sv - t_lo, -1, SEG_W) + 8
            r_prev, acc = carry
            for j in range(NLANE):
                r = rv[j]
                i = gbase + j
                diff = r != r_prev

                @pl.when(diff)
                def _flush(r_prev=r_prev, acc=acc):
                    for d in range(DGRP):
                        sl = pl.ds(d * NLANE, NLANE)
                        slab[r_prev, sl] = jnp.maximum(slab[r_prev, sl], acc[d])

                acc = tuple(
                    jnp.maximum(
                        jnp.where(diff, neg_inf, acc[d]),
                        rowsv[i, pl.ds(d * NLANE, NLANE)],
                    )
                    for d in range(DGRP)
                )
                r_prev = r
            return r_prev, acc

        return lax.fori_loop(0, C // NLANE, mem_body, carry)

    acc0 = tuple(neg_inf for _ in range(DGRP))
    r_prev, acc = lax.fori_loop(0, nchunks, chunk_body, (jnp.int32(0), acc0))
    for d in range(DGRP):
        sl = pl.ds(d * NLANE, NLANE)
        slab[r_prev, sl] = jnp.maximum(slab[r_prev, sl], acc[d])

    pltpu.sync_copy(slab.at[pl.ds(8, SEG_W)], out_hbm.at[pl.ds(t_lo, SEG_W)])


@functools.partial(
    pl.kernel,
    out_type=jax.ShapeDtypeStruct((H, D), jnp.float32),
    mesh=_MESH,
    scratch_types=[
        pltpu.VMEM((16,), jnp.int32),
        pltpu.VMEM((16,), jnp.int32),
        pltpu.VMEM((C,), jnp.int32),
        pltpu.VMEM((C,), jnp.int32),
        pltpu.VMEM((C, D), jnp.float32),
        pltpu.VMEM((SLAB_ROWS, D), jnp.float32),
        pltpu.SemaphoreType.DMA,
    ],
    compiler_params=pltpu.CompilerParams(needs_layout_passes=False),
)
def _hyperedge_max(emb_hbm, ids_hbm, seg_hbm, out_hbm,
                   idx16, segv16, idsv, segv, rowsv, slab, sem):
    _sc_body(emb_hbm, ids_hbm, seg_hbm, out_hbm,
             idx16, segv16, idsv, segv, rowsv, slab, sem)


@jax.jit
def kernel(embedding_table, flat_node_ids, segment_ids):
    return _hyperedge_max(embedding_table, flat_node_ids, segment_ids)


# speedup vs baseline: 6.4642x; 1.4069x over previous
import functools

import jax
import jax.numpy as jnp
from jax import lax
from jax.experimental import pallas as pl
from jax.experimental.pallas import tpu as pltpu
from jax.experimental.pallas import tpu_sc as plsc

N_NODES = 10000
D = 128
E = 320000
H = 20000

NUM_WORKERS = 32
SEG_W = 632
SLAB_ROWS = SEG_W + 16
C = 128
NLANE = 16
DGRP = D // NLANE

_MESH = plsc.VectorSubcoreMesh(core_axis_name="c", subcore_axis_name="s")


def _lower_bound(seg_hbm, target, idx16, segv16, sem):
    lanes = jnp.arange(16, dtype=jnp.int32)

    def rnd(_, carry):
        lo, n = carry
        step = (n + 15) // 16
        probes = lo + (lanes + 1) * step - 1
        hi_abs = jnp.minimum(lo + n, E)
        valid = probes < hi_abs
        idx16[...] = jnp.minimum(probes, E - 1)
        pltpu.async_copy(seg_hbm.at[idx16], segv16, sem).wait()
        less = valid & (segv16[...] < target)
        cnt = plsc.all_reduce_population_count(less)[0]
        return lo + cnt * step, step

    lo, _ = lax.fori_loop(0, 6, rnd, (jnp.int32(0), jnp.int32(E)))
    return lo


def _sc_body(emb_hbm, ids_hbm, seg_hbm, out_hbm,
             idx16, segv16, idsv, segv, rowsv, idsb, segb, rowsb, slab,
             sem, semb):
    wid = lax.axis_index("s") * 2 + lax.axis_index("c")
    t_lo = jnp.minimum(wid * SEG_W, H - SEG_W)

    lo = _lower_bound(seg_hbm, t_lo, idx16, segv16, sem)
    hi = _lower_bound(seg_hbm, t_lo + SEG_W, idx16, segv16, sem)
    a_lo = (lo // 8) * 8
    a_hi = jnp.minimum(((hi + 7) // 8) * 8, E)
    nchunks = (a_hi - a_lo + C - 1) // C

    neg_inf = jnp.full((NLANE,), -jnp.inf, dtype=jnp.float32)

    def init_body(r, _):
        for d in range(DGRP):
            slab[r, pl.ds(d * NLANE, NLANE)] = neg_inf
        return 0

    lax.fori_loop(0, SLAB_ROWS, init_body, 0)

    def fetch(c, ids_ref, seg_ref, rows_ref, dsem):
        base = jnp.minimum(a_lo + c * C, E - C)
        pltpu.sync_copy(seg_hbm.at[pl.ds(base, C)], seg_ref)
        pltpu.sync_copy(ids_hbm.at[pl.ds(base, C)], ids_ref)
        pltpu.async_copy(emb_hbm.at[ids_ref], rows_ref, dsem)

    def wait_rows(ids_ref, rows_ref, dsem):
        pltpu.make_async_copy(emb_hbm.at[ids_ref], rows_ref, dsem).wait()

    def process(seg_ref, rows_ref, carry):
        def mem_body(g, carry):
            gbase = g * NLANE
            sv = seg_ref[pl.ds(gbase, NLANE)]
            rv = jnp.clip(sv - t_lo, -1, SEG_W) + 8
            r_prev, acc = carry
            for j in range(NLANE):
                r = rv[j]
                i = gbase + j
                diff = r != r_prev

                @pl.when(diff)
                def _flush(r_prev=r_prev, acc=acc):
                    for d in range(DGRP):
                        sl = pl.ds(d * NLANE, NLANE)
                        slab[r_prev, sl] = jnp.maximum(slab[r_prev, sl], acc[d])

                acc = tuple(
                    jnp.maximum(
                        jnp.where(diff, neg_inf, acc[d]),
                        rows_ref[i, pl.ds(d * NLANE, NLANE)],
                    )
                    for d in range(DGRP)
                )
                r_prev = r
            return r_prev, acc

        return lax.fori_loop(0, C // NLANE, mem_body, carry)

    @pl.when(nchunks > 0)
    def _prologue():
        fetch(0, idsv, segv, rowsv, sem)

    npairs = (nchunks + 1) // 2

    def pair_body(p, carry):
        c1 = 2 * p + 1

        @pl.when(c1 < nchunks)
        def _fetch_b():
            fetch(c1, idsb, segb, rowsb, semb)

        wait_rows(idsv, rowsv, sem)
        carry = process(segv, rowsv, carry)

        @pl.when(c1 + 1 < nchunks)
        def _fetch_a():
            fetch(c1 + 1, idsv, segv, rowsv, sem)

        def do_b(carry):
            wait_rows(idsb, rowsb, semb)
            return process(segb, rowsb, carry)

        return lax.cond(c1 < nchunks, do_b, lambda carry: carry, carry)

    acc0 = tuple(neg_inf for _ in range(DGRP))
    r_prev, acc = lax.fori_loop(0, npairs, pair_body, (jnp.int32(0), acc0))
    for d in range(DGRP):
        sl = pl.ds(d * NLANE, NLANE)
        slab[r_prev, sl] = jnp.maximum(slab[r_prev, sl], acc[d])

    pltpu.sync_copy(slab.at[pl.ds(8, SEG_W)], out_hbm.at[pl.ds(t_lo, SEG_W)])


@functools.partial(
    pl.kernel,
    out_type=jax.ShapeDtypeStruct((H, D), jnp.float32),
    mesh=_MESH,
    scratch_types=[
        pltpu.VMEM((16,), jnp.int32),
        pltpu.VMEM((16,), jnp.int32),
        pltpu.VMEM((C,), jnp.int32),
        pltpu.VMEM((C,), jnp.int32),
        pltpu.VMEM((C, D), jnp.float32),
        pltpu.VMEM((C,), jnp.int32),
        pltpu.VMEM((C,), jnp.int32),
        pltpu.VMEM((C, D), jnp.float32),
        pltpu.VMEM((SLAB_ROWS, D), jnp.float32),
        pltpu.SemaphoreType.DMA,
        pltpu.SemaphoreType.DMA,
    ],
    compiler_params=pltpu.CompilerParams(needs_layout_passes=False),
)
def _hyperedge_max(emb_hbm, ids_hbm, seg_hbm, out_hbm,
                   idx16, segv16, idsv, segv, rowsv, idsb, segb, rowsb,
                   slab, sem, semb):
    _sc_body(emb_hbm, ids_hbm, seg_hbm, out_hbm,
             idx16, segv16, idsv, segv, rowsv, idsb, segb, rowsb, slab,
             sem, semb)


@jax.jit
def kernel(embedding_table, flat_node_ids, segment_ids):
    return _hyperedge_max(embedding_table, flat_node_ids, segment_ids)


# speedup vs baseline: 8.0595x; 1.2468x over previous
import functools

import jax
import jax.numpy as jnp
from jax import lax
from jax.experimental import pallas as pl
from jax.experimental.pallas import tpu as pltpu
from jax.experimental.pallas import tpu_sc as plsc

N_NODES = 10000
D = 128
E = 320000
H = 20000

NUM_WORKERS = 32
SEG_W = 632
SLAB_ROWS = SEG_W + 16
C = 128
NLANE = 16
DGRP = D // NLANE

_MESH = plsc.VectorSubcoreMesh(core_axis_name="c", subcore_axis_name="s")


def _dual_lower_bound(seg_hbm, t1, t2, idx16, segv16, idx16b, segv16b,
                      sem, semb):
    lanes = jnp.arange(16, dtype=jnp.int32)

    def rnd(_, carry):
        lo1, lo2, n = carry
        step = (n + 15) // 16
        p1 = lo1 + (lanes + 1) * step - 1
        p2 = lo2 + (lanes + 1) * step - 1
        idx16[...] = jnp.minimum(p1, E - 1)
        idx16b[...] = jnp.minimum(p2, E - 1)
        h1 = pltpu.async_copy(seg_hbm.at[idx16], segv16, sem)
        h2 = pltpu.async_copy(seg_hbm.at[idx16b], segv16b, semb)
        h1.wait()
        h2.wait()
        less1 = (p1 < jnp.minimum(lo1 + n, E)) & (segv16[...] < t1)
        less2 = (p2 < jnp.minimum(lo2 + n, E)) & (segv16b[...] < t2)
        cnt1 = plsc.all_reduce_population_count(less1)[0]
        cnt2 = plsc.all_reduce_population_count(less2)[0]
        return lo1 + cnt1 * step, lo2 + cnt2 * step, step

    return lax.fori_loop(
        0, 4, rnd, (jnp.int32(0), jnp.int32(0), jnp.int32(E)))


def _sc_body(emb_hbm, ids_hbm, seg_hbm, out_hbm,
             idx16, segv16, idx16b, segv16b, idsv, segv, rowsv,
             idsb, segb, rowsb, slab, sem, semb):
    wid = lax.axis_index("s") * 2 + lax.axis_index("c")
    t_lo = jnp.minimum(wid * SEG_W, H - SEG_W)

    lo, hi, slack = _dual_lower_bound(
        seg_hbm, t_lo, t_lo + SEG_W, idx16, segv16, idx16b, segv16b,
        sem, semb)
    a_lo = (lo // 8) * 8
    a_hi = jnp.minimum(((hi + slack + 7) // 8) * 8, E)
    nchunks = (a_hi - a_lo + C - 1) // C

    neg_inf = jnp.full((NLANE,), -jnp.inf, dtype=jnp.float32)

    def init_body(r, _):
        for d in range(DGRP):
            slab[r, pl.ds(d * NLANE, NLANE)] = neg_inf
        return 0

    lax.fori_loop(0, SLAB_ROWS, init_body, 0)

    def chunk_skip(c):
        base = jnp.minimum(a_lo + c * C, E - C)
        return (a_lo + c * C) - base

    def fetch(c, ids_ref, seg_ref, rows_ref, dsem):
        base = jnp.minimum(a_lo + c * C, E - C)
        pltpu.sync_copy(seg_hbm.at[pl.ds(base, C)], seg_ref)
        pltpu.sync_copy(ids_hbm.at[pl.ds(base, C)], ids_ref)
        pltpu.async_copy(emb_hbm.at[ids_ref], rows_ref, dsem)

    def wait_rows(ids_ref, rows_ref, dsem):
        pltpu.make_async_copy(emb_hbm.at[ids_ref], rows_ref, dsem).wait()

    lanes16 = jnp.arange(16, dtype=jnp.int32)

    def process(seg_ref, rows_ref, skip, carry):
        def mem_body(g, carry):
            gbase = g * NLANE
            sv = seg_ref[pl.ds(gbase, NLANE)]
            rv = jnp.clip(sv - t_lo, -1, SEG_W) + 8
            rv = jnp.where(gbase + lanes16 < skip, 7, rv)
            r_prev, acc = carry
            for j in range(NLANE):
                r = rv[j]
                i = gbase + j
                diff = r != r_prev
                acc = tuple(
                    jnp.maximum(
                        jnp.where(diff, neg_inf, acc[d]),
                        rows_ref[i, pl.ds(d * NLANE, NLANE)],
                    )
                    for d in range(DGRP)
                )
                for d in range(DGRP):
                    slab[r, pl.ds(d * NLANE, NLANE)] = acc[d]
                r_prev = r
            return r_prev, acc

        return lax.fori_loop(0, C // NLANE, mem_body, carry)

    @pl.when(nchunks > 0)
    def _prologue():
        fetch(0, idsv, segv, rowsv, sem)

    npairs = (nchunks + 1) // 2

    def pair_body(p, carry):
        c1 = 2 * p + 1

        @pl.when(c1 < nchunks)
        def _fetch_b():
            fetch(c1, idsb, segb, rowsb, semb)

        wait_rows(idsv, rowsv, sem)
        carry = process(segv, rowsv, chunk_skip(2 * p), carry)

        @pl.when(c1 + 1 < nchunks)
        def _fetch_a():
            fetch(c1 + 1, idsv, segv, rowsv, sem)

        def do_b(carry):
            wait_rows(idsb, rowsb, semb)
            return process(segb, rowsb, chunk_skip(c1), carry)

        return lax.cond(c1 < nchunks, do_b, lambda carry: carry, carry)

    acc0 = tuple(neg_inf for _ in range(DGRP))
    lax.fori_loop(0, npairs, pair_body, (jnp.int32(0), acc0))

    pltpu.sync_copy(slab.at[pl.ds(8, SEG_W)], out_hbm.at[pl.ds(t_lo, SEG_W)])


@functools.partial(
    pl.kernel,
    out_type=jax.ShapeDtypeStruct((H, D), jnp.float32),
    mesh=_MESH,
    scratch_types=[
        pltpu.VMEM((16,), jnp.int32),
        pltpu.VMEM((16,), jnp.int32),
        pltpu.VMEM((16,), jnp.int32),
        pltpu.VMEM((16,), jnp.int32),
        pltpu.VMEM((C,), jnp.int32),
        pltpu.VMEM((C,), jnp.int32),
        pltpu.VMEM((C, D), jnp.float32),
        pltpu.VMEM((C,), jnp.int32),
        pltpu.VMEM((C,), jnp.int32),
        pltpu.VMEM((C, D), jnp.float32),
        pltpu.VMEM((SLAB_ROWS, D), jnp.float32),
        pltpu.SemaphoreType.DMA,
        pltpu.SemaphoreType.DMA,
    ],
    compiler_params=pltpu.CompilerParams(needs_layout_passes=False),
)
def _hyperedge_max(emb_hbm, ids_hbm, seg_hbm, out_hbm,
                   idx16, segv16, idx16b, segv16b, idsv, segv, rowsv,
                   idsb, segb, rowsb, slab, sem, semb):
    _sc_body(emb_hbm, ids_hbm, seg_hbm, out_hbm,
             idx16, segv16, idx16b, segv16b, idsv, segv, rowsv,
             idsb, segb, rowsb, slab, sem, semb)


@jax.jit
def kernel(embedding_table, flat_node_ids, segment_ids):
    return _hyperedge_max(embedding_table, flat_node_ids, segment_ids)
